# baseline (speedup 1.0000x reference)
import jax
import jax.numpy as jnp
from jax import lax
from jax.experimental import pallas as pl
from jax.experimental.pallas import tpu as pltpu
import jax.experimental.pallas.tpu_sc as plsc

EMBED = 64
B = 4096
L = 20
PAIR = 2 * EMBED
TBL_ROWS = 1000000
TBL_PAIRS = TBL_ROWS // 2

NC = 2
NS = 16
NW = NC * NS
EX_PER_W = B // NW
CHUNK = 32
N_CHUNK = EX_PER_W // CHUNK
ROWS_PER_CHUNK = CHUNK * L
IDX_COLS = 128
IDX_ROWS = ROWS_PER_CHUNK // IDX_COLS


def _gather_jobs(jobs, idx_v, rows_v, sem):
    wid = lax.axis_index("s") * NC + lax.axis_index("c")

    for tbl, idx_hbm, out_hbm in jobs:
        def chunk_body(cidx, _, tbl=tbl, idx_hbm=idx_hbm, out_hbm=out_hbm):
            r0 = (wid * EX_PER_W + cidx * CHUNK) * L
            pltpu.sync_copy(idx_hbm.at[pl.ds(r0, ROWS_PER_CHUNK)], idx_v)
            for j in range(IDX_ROWS):
                pltpu.async_copy(tbl.at[idx_v.at[pl.ds(j * IDX_COLS, IDX_COLS)]],
                                 rows_v.at[pl.ds(j * IDX_COLS, IDX_COLS)], sem)
            for j in range(IDX_ROWS):
                pltpu.make_async_copy(tbl.at[idx_v.at[pl.ds(j * IDX_COLS, IDX_COLS)]],
                                      rows_v.at[pl.ds(j * IDX_COLS, IDX_COLS)],
                                      sem).wait()
            pltpu.sync_copy(rows_v, out_hbm.at[pl.ds(r0, ROWS_PER_CHUNK)])
            return 0

        lax.fori_loop(0, N_CHUNK, chunk_body, 0)


def _sc_all_body(mnt_mat, ent_mat, mnt_idx, pos_idx, neg_idx,
                 out_m, out_p, out_n, idx_v, rows_v, sem):
    _gather_jobs(((mnt_mat, mnt_idx, out_m),
                  (ent_mat, pos_idx, out_p),
                  (ent_mat, neg_idx, out_n)), idx_v, rows_v, sem)


_SC_SCRATCH = [
    pltpu.VMEM((ROWS_PER_CHUNK,), jnp.int32),
    pltpu.VMEM((ROWS_PER_CHUNK, PAIR), jnp.float32),
    pltpu.SemaphoreType.DMA,
]

_sc_all = pl.kernel(
    _sc_all_body,
    out_type=(jax.ShapeDtypeStruct((B * L, PAIR), jnp.float32),
              jax.ShapeDtypeStruct((B * L, PAIR), jnp.float32),
              jax.ShapeDtypeStruct((B * L, PAIR), jnp.float32)),
    mesh=plsc.VectorSubcoreMesh(core_axis_name="c", subcore_axis_name="s"),
    scratch_types=_SC_SCRATCH,
)


def _tc_finish_body(gm_ref, gp_ref, gn_ref, hm_ref, hp_ref, hn_ref,
                    mb_ref, eb_ref, sp_ref, sn_ref):
    def pick(g_ref, h_ref):
        g = g_ref[...]
        h = h_ref[...][..., None]
        return jnp.where(h == 1, g[:, :, EMBED:], g[:, :, :EMBED])

    m = jnp.tanh(jnp.sum(pick(gm_ref, hm_ref), axis=1) + mb_ref[...])
    p = jnp.tanh(jnp.sum(pick(gp_ref, hp_ref), axis=1) + eb_ref[...])
    n = jnp.tanh(jnp.sum(pick(gn_ref, hn_ref), axis=1) + eb_ref[...])
    eps = 1e-12
    rm = lax.rsqrt(jnp.maximum(jnp.sum(m * m, axis=1), eps))
    rp = lax.rsqrt(jnp.maximum(jnp.sum(p * p, axis=1), eps))
    rn = lax.rsqrt(jnp.maximum(jnp.sum(n * n, axis=1), eps))
    mp = jnp.sum(m * p, axis=1)
    mn = jnp.sum(m * n, axis=1)
    sp_ref[...] = -(mp * rm * rp)
    sn_ref[...] = -(mn * rm * rn)


_TC_BLK = 256


def _tc_finish(gm, gp, gn, hm, hp, hn, mb, eb):
    grid = B // _TC_BLK
    g_spec = pl.BlockSpec((_TC_BLK, L, PAIR), lambda i: (i, 0, 0))
    h_spec = pl.BlockSpec((_TC_BLK, L), lambda i: (i, 0))
    bias_spec = pl.BlockSpec((1, EMBED), lambda i: (0, 0))
    out_spec = pl.BlockSpec((_TC_BLK,), lambda i: (i,))
    return pl.pallas_call(
        _tc_finish_body,
        grid=(grid,),
        in_specs=[g_spec, g_spec, g_spec, h_spec, h_spec, h_spec,
                  bias_spec, bias_spec],
        out_specs=[out_spec, out_spec],
        out_shape=[jax.ShapeDtypeStruct((B,), jnp.float32),
                   jax.ShapeDtypeStruct((B,), jnp.float32)],
    )(gm, gp, gn, hm, hp, hn, mb, eb)


def kernel(mention_idx, ent_pos_idx, ent_neg_idx, mnt_matrix, ent_matrix,
           mnt_bias, ent_bias):
    mi = mention_idx.astype(jnp.int32).reshape(B * L)
    pi = ent_pos_idx.astype(jnp.int32).reshape(B * L)
    ni = ent_neg_idx.astype(jnp.int32).reshape(B * L)
    gm, gp, gn = _sc_all(mnt_matrix.reshape(TBL_PAIRS, PAIR),
                         ent_matrix.reshape(TBL_PAIRS, PAIR),
                         mi >> 1, pi >> 1, ni >> 1)
    sp, sn = _tc_finish(gm.reshape(B, L, PAIR), gp.reshape(B, L, PAIR),
                        gn.reshape(B, L, PAIR),
                        (mi & 1).reshape(B, L), (pi & 1).reshape(B, L),
                        (ni & 1).reshape(B, L),
                        mnt_bias.reshape(1, EMBED), ent_bias.reshape(1, EMBED))
    return sp, sn
